# baseline (speedup 1.0000x reference)
import functools

import jax
import jax.numpy as jnp
from jax import lax
from jax.experimental import pallas as pl
from jax.experimental.pallas import tpu as pltpu
from jax.experimental.pallas import tpu_sc as plsc

NC = 2
NS = 16
L = 16
NW = NC * NS


def _matmul_body(x_ref, w_ref, o_ref):
    o_ref[...] = jnp.dot(x_ref[...], w_ref[...],
                         preferred_element_type=jnp.float32
                         ).astype(jnp.bfloat16)


def _support_matmul(x, W):
    n, d = x.shape
    blk = 2000
    grid = n // blk
    return pl.pallas_call(
        _matmul_body,
        grid=(grid,),
        in_specs=[
            pl.BlockSpec((blk, d), lambda i: (i, 0)),
            pl.BlockSpec((d, d), lambda i: (0, 0)),
        ],
        out_specs=pl.BlockSpec((blk, d), lambda i: (i, 0)),
        out_shape=jax.ShapeDtypeStruct((n, d), jnp.bfloat16),
    )(x, W)


def _combine_body(p_ref, b_ref, o_ref):
    o_ref[...] = p_ref[0] + p_ref[1] + b_ref[...]


def _combine(partials, b):
    _, n, d = partials.shape
    blk = 1000
    grid = n // blk
    return pl.pallas_call(
        _combine_body,
        grid=(grid,),
        in_specs=[
            pl.BlockSpec((2, blk, d), lambda i: (0, i, 0)),
            pl.BlockSpec((d,), lambda i: (0,)),
        ],
        out_specs=pl.BlockSpec((blk, d), lambda i: (i, 0)),
        out_shape=jax.ShapeDtypeStruct((n, d), jnp.float32),
    )(partials, b)


R = 2
DP = 200


def _lane_bcast(v, t):
    idx = jnp.full((L, 1), t, jnp.int32)
    dn = lax.GatherDimensionNumbers(
        offset_dims=(), collapsed_slice_dims=(0,), start_index_map=(0,))
    return lax.gather(v, idx, dn, (1,),
                      mode=lax.GatherScatterMode.PROMISE_IN_BOUNDS)


def _spmm_sc(support, zrs, src2, dst2, w2, n, d, nchunks, k, per_w):
    npieces = n // DP
    groups = nchunks // R

    mesh = plsc.VectorSubcoreMesh(core_axis_name="c", subcore_axis_name="s")

    @functools.partial(
        pl.kernel,
        out_type=jax.ShapeDtypeStruct((NC, n, d), jnp.float32),
        mesh=mesh,
        scratch_types=[
            pltpu.VMEM((per_w,), jnp.int32),
            pltpu.VMEM((per_w,), jnp.int32),
            pltpu.VMEM((per_w,), jnp.float32),
            pltpu.VMEM((R * k, d // 2), jnp.int32),
            pltpu.VMEM((R * k, d), jnp.float32),
            pltpu.VMEM_SHARED((n, d), jnp.float32),
            [pltpu.SemaphoreType.DMA] * R,
            [pltpu.SemaphoreType.DMA] * R,
        ],
        compiler_params=pltpu.CompilerParams(needs_layout_passes=False,
                                             use_tc_tiling_on_sc=False),
    )
    def spmm(sup_hbm, zrs_hbm, src_hbm, dst_hbm, w_hbm, out_hbm,
             src_v, dst_v, w_v, gbuf, sbuf, acc, gsems, ssems):
        c = lax.axis_index("c")
        s = lax.axis_index("s")
        wid = c * NS + s

        pltpu.sync_copy(src_hbm.at[wid], src_v)
        pltpu.sync_copy(dst_hbm.at[wid], dst_v)
        pltpu.sync_copy(w_hbm.at[wid], w_v)

        def zero_piece(i, carry):
            p = i * NS + s

            @pl.when(p < npieces)
            def _():
                sl = pl.ds(p * DP, DP)
                pltpu.sync_copy(zrs_hbm.at[sl], acc.at[sl])
            return carry

        lax.fori_loop(0, (npieces + NS - 1) // NS, zero_piece, 0)
        plsc.subcore_barrier()

        for b in range(R):
            pltpu.async_copy(sup_hbm.at[src_v.at[pl.ds(b * k, k)]],
                             gbuf.at[pl.ds(b * k, k)], gsems[b])

        def scatter_chunk(ci, b, add, wait_only):
            for sub in range(k // L):
                idx = dst_v[pl.ds(ci * k + sub * L, L)]
                src_sl = sbuf.at[pl.ds(b * k + sub * L, L)]
                if wait_only:
                    pltpu.make_async_copy(src_sl, acc.at[idx],
                                          ssems[b]).wait()
                else:
                    pltpu.async_copy(src_sl, acc.at[idx], ssems[b], add=add)

        def do_group(i, carry):
            for b in range(R):
                ci = i * R + b

                @pl.when(ci >= R)
                def _():
                    scatter_chunk(ci, b, True, wait_only=True)

                pltpu.make_async_copy(
                    sup_hbm.at[src_v.at[pl.ds(ci * k, k)]],
                    gbuf.at[pl.ds(b * k, k)], gsems[b]).wait()

                def scale_group(g, inner):
                    wv = w_v[pl.ds(ci * k + g * L, L)]
                    for t in range(L):
                        ws = _lane_bcast(wv, t)
                        r = b * k + g * L + t
                        for j in range(d // (2 * L)):
                            x = gbuf[r, pl.ds(j * L, L)]
                            lo = plsc.bitcast(x << 16, jnp.float32)
                            hi = plsc.bitcast(x & jnp.int32(-65536),
                                              jnp.float32)
                            sbuf[r, pl.ds(j * 2 * L, L)] = lo * ws
                            sbuf[r, pl.ds(j * 2 * L + L, L)] = hi * ws
                    return inner

                lax.fori_loop(0, k // L, scale_group, 0)

                scatter_chunk(ci, b, True, wait_only=False)

                @pl.when(ci + R < nchunks)
                def _():
                    pltpu.async_copy(
                        sup_hbm.at[src_v.at[pl.ds((ci + R) * k, k)]],
                        gbuf.at[pl.ds(b * k, k)], gsems[b])
            return carry

        lax.fori_loop(0, groups, do_group, 0)

        ntail = per_w - nchunks * k
        if ntail:
            assert ntail == L
            base = nchunks * k
            pltpu.async_copy(sup_hbm.at[src_v.at[pl.ds(base, L)]],
                             gbuf.at[pl.ds(0, L)], gsems[0]).wait()
            wv = w_v[pl.ds(base, L)]
            for t_ in range(L):
                ws = _lane_bcast(wv, t_)
                for j in range(d // (2 * L)):
                    x = gbuf[t_, pl.ds(j * L, L)]
                    lo = plsc.bitcast(x << 16, jnp.float32)
                    hi = plsc.bitcast(x & jnp.int32(-65536), jnp.float32)
                    sbuf[t_, pl.ds(j * 2 * L, L)] = lo * ws
                    sbuf[t_, pl.ds(j * 2 * L + L, L)] = hi * ws
            idx = dst_v[pl.ds(base, L)]
            pltpu.async_copy(sbuf.at[pl.ds(0, L)], acc.at[idx], ssems[0],
                             add=True).wait()

        for b in range(R):
            scatter_chunk(b, b, True, wait_only=True)
        plsc.subcore_barrier()

        def drain_piece(i, carry):
            p = i * NS + s

            @pl.when(p < npieces)
            def _():
                sl = pl.ds(p * DP, DP)
                pltpu.sync_copy(acc.at[sl], out_hbm.at[c, sl])
            return carry

        lax.fori_loop(0, (npieces + NS - 1) // NS, drain_piece, 0)

    return spmm(support, zrs, src2, dst2, w2)


def kernel(input, edge_index, edge_weight, W, b):
    n, d = input.shape
    e = edge_weight.shape[0]
    k = 32
    per_w = e // NW
    nchunks = per_w // (R * k) * R

    m = jnp.arange(d)
    colperm = (m // 32) * 32 + (m % 32) // 2 + 16 * (m % 2)
    support = _support_matmul(input, W[:, colperm])
    support = jax.lax.bitcast_convert_type(
        support.reshape(n, d // 2, 2), jnp.int32)
    zrs = jnp.zeros((n, d), jnp.float32)

    src2 = edge_index[0].reshape(NW, per_w)
    dst2 = edge_index[1].reshape(NW, per_w)
    w2 = edge_weight.reshape(NW, per_w)

    partials = _spmm_sc(support, zrs, src2, dst2, w2, n, d, nchunks, k,
                        per_w)
    return _combine(partials, b)


# speedup vs baseline: 1.4740x; 1.4740x over previous
import functools

import jax
import jax.numpy as jnp
from jax import lax
from jax.experimental import pallas as pl
from jax.experimental.pallas import tpu as pltpu
from jax.experimental.pallas import tpu_sc as plsc

NC = 2
NS = 16
L = 16
NW = NC * NS


def _matmul_body(x_ref, w_ref, o_ref):
    o_ref[...] = jnp.dot(x_ref[...], w_ref[...],
                         preferred_element_type=jnp.float32)


def _support_matmul(x, W):
    n, d = x.shape
    blk = 1000
    grid = n // blk
    return pl.pallas_call(
        _matmul_body,
        grid=(grid,),
        in_specs=[
            pl.BlockSpec((blk, d), lambda i: (i, 0)),
            pl.BlockSpec((d, d), lambda i: (0, 0)),
        ],
        out_specs=pl.BlockSpec((blk, d), lambda i: (i, 0)),
        out_shape=jax.ShapeDtypeStruct((n, d), jnp.float32),
    )(x, W)


def _combine_body(p_ref, b_ref, o_ref):
    o_ref[...] = p_ref[0] + p_ref[1] + b_ref[...]


def _combine(partials, b):
    _, n, d = partials.shape
    blk = 1000
    grid = n // blk
    return pl.pallas_call(
        _combine_body,
        grid=(grid,),
        in_specs=[
            pl.BlockSpec((2, blk, d), lambda i: (0, i, 0)),
            pl.BlockSpec((d,), lambda i: (0,)),
        ],
        out_specs=pl.BlockSpec((blk, d), lambda i: (i, 0)),
        out_shape=jax.ShapeDtypeStruct((n, d), jnp.float32),
    )(partials, b)


R = 2
DP = 200


def _lane_bcast(v, t):
    idx = jnp.full((L, 1), t, jnp.int32)
    dn = lax.GatherDimensionNumbers(
        offset_dims=(), collapsed_slice_dims=(0,), start_index_map=(0,))
    return lax.gather(v, idx, dn, (1,),
                      mode=lax.GatherScatterMode.PROMISE_IN_BOUNDS)


def _spmm_sc(support, zrs, src2, dst2, w2, n, d, nchunks, k, per_w):
    npieces = n // DP
    groups = nchunks // R

    mesh = plsc.VectorSubcoreMesh(core_axis_name="c", subcore_axis_name="s")

    @functools.partial(
        pl.kernel,
        out_type=jax.ShapeDtypeStruct((NC, n, d), jnp.float32),
        mesh=mesh,
        scratch_types=[
            pltpu.VMEM((per_w,), jnp.int32),
            pltpu.VMEM((per_w,), jnp.int32),
            pltpu.VMEM((per_w,), jnp.float32),
            pltpu.VMEM((R * k, d), jnp.float32),
            pltpu.VMEM((R * k, d), jnp.float32),
            pltpu.VMEM_SHARED((n, d), jnp.float32),
            [pltpu.SemaphoreType.DMA] * R,
            [pltpu.SemaphoreType.DMA] * R,
        ],
        compiler_params=pltpu.CompilerParams(needs_layout_passes=False),
    )
    def spmm(sup_hbm, zrs_hbm, src_hbm, dst_hbm, w_hbm, out_hbm,
             src_v, dst_v, w_v, gbuf, sbuf, acc, gsems, ssems):
        c = lax.axis_index("c")
        s = lax.axis_index("s")
        wid = c * NS + s

        pltpu.sync_copy(src_hbm.at[wid], src_v)
        pltpu.sync_copy(dst_hbm.at[wid], dst_v)
        pltpu.sync_copy(w_hbm.at[wid], w_v)

        def zero_piece(i, carry):
            p = i * NS + s

            @pl.when(p < npieces)
            def _():
                sl = pl.ds(p * DP, DP)
                pltpu.sync_copy(zrs_hbm.at[sl], acc.at[sl])
            return carry

        lax.fori_loop(0, (npieces + NS - 1) // NS, zero_piece, 0)
        plsc.subcore_barrier()

        for b in range(R):
            pltpu.async_copy(sup_hbm.at[src_v.at[pl.ds(b * k, k)]],
                             gbuf.at[pl.ds(b * k, k)], gsems[b])

        def scatter_chunk(ci, b, add, wait_only):
            idx = dst_v.at[pl.ds(ci * k, k)]
            src_sl = sbuf.at[pl.ds(b * k, k)]
            if wait_only:
                pltpu.make_async_copy(src_sl, acc.at[idx], ssems[b]).wait()
            else:
                pltpu.async_copy(src_sl, acc.at[idx], ssems[b], add=add)

        def do_group(i, carry):
            for b in range(R):
                ci = i * R + b

                @pl.when(ci >= R)
                def _():
                    scatter_chunk(ci, b, True, wait_only=True)

                pltpu.make_async_copy(
                    sup_hbm.at[src_v.at[pl.ds(ci * k, k)]],
                    gbuf.at[pl.ds(b * k, k)], gsems[b]).wait()

                def scale_group(g, inner):
                    wv = w_v[pl.ds(ci * k + g * L, L)]
                    for t in range(L):
                        ws = _lane_bcast(wv, t)
                        r = b * k + g * L + t
                        for j in range(d // L):
                            sl = pl.ds(j * L, L)
                            sbuf[r, sl] = gbuf[r, sl] * ws
                    return inner

                lax.fori_loop(0, k // L, scale_group, 0)

                scatter_chunk(ci, b, True, wait_only=False)

                @pl.when(ci + R < nchunks)
                def _():
                    pltpu.async_copy(
                        sup_hbm.at[src_v.at[pl.ds((ci + R) * k, k)]],
                        gbuf.at[pl.ds(b * k, k)], gsems[b])
            return carry

        lax.fori_loop(0, groups, do_group, 0)

        ntail = per_w - nchunks * k
        if ntail:
            assert ntail == L
            base = nchunks * k
            pltpu.async_copy(sup_hbm.at[src_v.at[pl.ds(base, L)]],
                             gbuf.at[pl.ds(0, L)], gsems[0]).wait()
            wv = w_v[pl.ds(base, L)]
            for t_ in range(L):
                ws = _lane_bcast(wv, t_)
                for j in range(d // L):
                    sl = pl.ds(j * L, L)
                    sbuf[t_, sl] = gbuf[t_, sl] * ws
            idx = dst_v.at[pl.ds(base, L)]
            pltpu.async_copy(sbuf.at[pl.ds(0, L)], acc.at[idx], ssems[0],
                             add=True).wait()

        for b in range(R):
            scatter_chunk(b, b, True, wait_only=True)
        plsc.subcore_barrier()

        def drain_piece(i, carry):
            p = i * NS + s

            @pl.when(p < npieces)
            def _():
                sl = pl.ds(p * DP, DP)
                pltpu.sync_copy(acc.at[sl], out_hbm.at[c, sl])
            return carry

        lax.fori_loop(0, (npieces + NS - 1) // NS, drain_piece, 0)

    return spmm(support, zrs, src2, dst2, w2)


def kernel(input, edge_index, edge_weight, W, b):
    n, d = input.shape
    e = edge_weight.shape[0]
    k = 32
    per_w = e // NW
    nchunks = per_w // (R * k) * R

    support = _support_matmul(input, W)
    zrs = jnp.zeros((n, d), jnp.float32)

    src2 = edge_index[0].reshape(NW, per_w)
    dst2 = edge_index[1].reshape(NW, per_w)
    w2 = edge_weight.reshape(NW, per_w)

    partials = _spmm_sc(support, zrs, src2, dst2, w2, n, d, nchunks, k,
                        per_w)
    return _combine(partials, b)


# speedup vs baseline: 1.8205x; 1.2351x over previous
import functools

import jax
import jax.numpy as jnp
from jax import lax
from jax.experimental import pallas as pl
from jax.experimental.pallas import tpu as pltpu
from jax.experimental.pallas import tpu_sc as plsc

NC = 2
NS = 16
L = 16
NW = NC * NS


def _matmul_body(x_ref, w_ref, o_ref):
    o_ref[...] = jnp.dot(x_ref[...], w_ref[...],
                         preferred_element_type=jnp.float32)


def _support_matmul(x, W):
    n, d = x.shape
    blk = 1000
    grid = n // blk
    return pl.pallas_call(
        _matmul_body,
        grid=(grid,),
        in_specs=[
            pl.BlockSpec((blk, d), lambda i: (i, 0)),
            pl.BlockSpec((d, d), lambda i: (0, 0)),
        ],
        out_specs=pl.BlockSpec((blk, d), lambda i: (i, 0)),
        out_shape=jax.ShapeDtypeStruct((n, d), jnp.float32),
    )(x, W)


def _combine_body(p_ref, b_ref, o_ref):
    o_ref[...] = p_ref[0] + p_ref[1] + b_ref[...]


def _combine(partials, b):
    _, n, d = partials.shape
    blk = 1000
    grid = n // blk
    return pl.pallas_call(
        _combine_body,
        grid=(grid,),
        in_specs=[
            pl.BlockSpec((2, blk, d), lambda i: (0, i, 0)),
            pl.BlockSpec((d,), lambda i: (0,)),
        ],
        out_specs=pl.BlockSpec((blk, d), lambda i: (i, 0)),
        out_shape=jax.ShapeDtypeStruct((n, d), jnp.float32),
    )(partials, b)


RG = 3
RS = 2
R = RG
DP = 200


def _lane_bcast(v, t):
    idx = jnp.full((L, 1), t, jnp.int32)
    dn = lax.GatherDimensionNumbers(
        offset_dims=(), collapsed_slice_dims=(0,), start_index_map=(0,))
    return lax.gather(v, idx, dn, (1,),
                      mode=lax.GatherScatterMode.PROMISE_IN_BOUNDS)


def _spmm_sc(support, zrs, src2, dst2, w2, n, d, nchunks, k, per_w):
    npieces = n // DP
    groups = nchunks // (RG * RS)

    mesh = plsc.VectorSubcoreMesh(core_axis_name="c", subcore_axis_name="s")

    @functools.partial(
        pl.kernel,
        out_type=jax.ShapeDtypeStruct((NC, n, d), jnp.float32),
        mesh=mesh,
        scratch_types=[
            pltpu.VMEM((per_w,), jnp.int32),
            pltpu.VMEM((per_w,), jnp.int32),
            pltpu.VMEM((per_w,), jnp.float32),
            pltpu.VMEM((RG * k, d), jnp.float32),
            pltpu.VMEM((RS * k, d), jnp.float32),
            pltpu.VMEM_SHARED((n, d), jnp.float32),
            [pltpu.SemaphoreType.DMA] * RG,
            [pltpu.SemaphoreType.DMA] * RS,
        ],
        compiler_params=pltpu.CompilerParams(needs_layout_passes=False),
    )
    def spmm(sup_hbm, zrs_hbm, src_hbm, dst_hbm, w_hbm, out_hbm,
             src_v, dst_v, w_v, gbuf, sbuf, acc, gsems, ssems):
        c = lax.axis_index("c")
        s = lax.axis_index("s")
        wid = c * NS + s

        pltpu.sync_copy(src_hbm.at[wid], src_v)
        pltpu.sync_copy(dst_hbm.at[wid], dst_v)
        pltpu.sync_copy(w_hbm.at[wid], w_v)

        def zero_piece(i, carry):
            p = i * NS + s

            @pl.when(p < npieces)
            def _():
                sl = pl.ds(p * DP, DP)
                pltpu.sync_copy(zrs_hbm.at[sl], acc.at[sl])
            return carry

        lax.fori_loop(0, (npieces + NS - 1) // NS, zero_piece, 0)
        plsc.subcore_barrier()

        for b in range(RG):
            pltpu.async_copy(sup_hbm.at[src_v.at[pl.ds(b * k, k)]],
                             gbuf.at[pl.ds(b * k, k)], gsems[b])

        def scatter_chunk(ci, b, add, wait_only):
            idx = dst_v.at[pl.ds(ci * k, k)]
            src_sl = sbuf.at[pl.ds(b * k, k)]
            if wait_only:
                pltpu.make_async_copy(src_sl, acc.at[idx], ssems[b]).wait()
            else:
                pltpu.async_copy(src_sl, acc.at[idx], ssems[b], add=add)

        def do_group(i, carry):
            for u in range(RG * RS):
                ci = i * (RG * RS) + u
                bg = u % RG
                bs = u % RS

                @pl.when(ci >= RS)
                def _():
                    scatter_chunk(ci, bs, True, wait_only=True)

                pltpu.make_async_copy(
                    sup_hbm.at[src_v.at[pl.ds(ci * k, k)]],
                    gbuf.at[pl.ds(bg * k, k)], gsems[bg]).wait()

                def scale_group(g, inner):
                    wv = w_v[pl.ds(ci * k + g * L, L)]
                    for t in range(L):
                        ws = _lane_bcast(wv, t)
                        for j in range(d // L):
                            sl = pl.ds(j * L, L)
                            sbuf[bs * k + g * L + t, sl] = (
                                gbuf[bg * k + g * L + t, sl] * ws)
                    return inner

                lax.fori_loop(0, k // L, scale_group, 0)

                scatter_chunk(ci, bs, True, wait_only=False)

                @pl.when(ci + RG < nchunks)
                def _():
                    pltpu.async_copy(
                        sup_hbm.at[src_v.at[pl.ds((ci + RG) * k, k)]],
                        gbuf.at[pl.ds(bg * k, k)], gsems[bg])
            return carry

        lax.fori_loop(0, groups, do_group, 0)

        for b in range(RS):
            scatter_chunk(b, b, True, wait_only=True)

        ntail = per_w - nchunks * k
        if ntail:
            assert ntail == L
            base = nchunks * k
            pltpu.async_copy(sup_hbm.at[src_v.at[pl.ds(base, L)]],
                             gbuf.at[pl.ds(0, L)], gsems[0]).wait()
            wv = w_v[pl.ds(base, L)]
            for t_ in range(L):
                ws = _lane_bcast(wv, t_)
                for j in range(d // L):
                    sl = pl.ds(j * L, L)
                    sbuf[t_, sl] = gbuf[t_, sl] * ws
            idx = dst_v.at[pl.ds(base, L)]
            pltpu.async_copy(sbuf.at[pl.ds(0, L)], acc.at[idx], ssems[0],
                             add=True).wait()

        plsc.subcore_barrier()

        def drain_piece(i, carry):
            p = i * NS + s

            @pl.when(p < npieces)
            def _():
                sl = pl.ds(p * DP, DP)
                pltpu.sync_copy(acc.at[sl], out_hbm.at[c, sl])
            return carry

        lax.fori_loop(0, (npieces + NS - 1) // NS, drain_piece, 0)

    return spmm(support, zrs, src2, dst2, w2)


def kernel(input, edge_index, edge_weight, W, b):
    n, d = input.shape
    e = edge_weight.shape[0]
    k = 32
    per_w = e // NW
    nchunks = per_w // (R * k) * R

    support = _support_matmul(input, W)
    zrs = jnp.zeros((n, d), jnp.float32)

    src2 = edge_index[0].reshape(NW, per_w)
    dst2 = edge_index[1].reshape(NW, per_w)
    w2 = edge_weight.reshape(NW, per_w)

    partials = _spmm_sc(support, zrs, src2, dst2, w2, n, d, nchunks, k,
                        per_w)
    return _combine(partials, b)
